# baseline (speedup 1.0000x reference)
import jax
import jax.numpy as jnp
from jax import lax
from jax.experimental import pallas as pl
from jax.experimental.pallas import tpu as pltpu
from jax.experimental.pallas import tpu_sc as plsc

B = 32
MAXLEN = 2048
D = 128
CHUNK = 64
NCHUNK = MAXLEN // CHUNK
NC = 2
NS = 16
NBT = 8
GDIST = 4
PDIST = 7


def _emb_body(x_hbm, tok_hbm, pos_hbm, out_hbm, idx_v, tok_v, pos_sh,
              sem_g, sem_p, sem_o, sem_i):
    c = lax.axis_index("c")
    s = lax.axis_index("s")
    w = s * NC + c

    idx_cp = pltpu.async_copy(x_hbm.at[w], idx_v, sem_i)

    def start_prefill(j):
        b = j % NBT
        pltpu.async_copy(pos_sh.at[j], tok_v.at[b], sem_p.at[b])

    def wait_prefill(j):
        b = j % NBT
        pltpu.make_async_copy(pos_sh.at[j], tok_v.at[b], sem_p.at[b]).wait()

    def start_gather(j):
        b = j % NBT
        pltpu.async_copy(tok_hbm.at[idx_v.at[j]], tok_v.at[b], sem_g.at[b], add=True)

    def wait_gather(j):
        b = j % NBT
        pltpu.make_async_copy(tok_hbm.at[idx_v.at[j]], tok_v.at[b], sem_g.at[b]).wait()

    def start_out(j):
        b = j % NBT
        pltpu.async_copy(tok_v.at[b], out_hbm.at[w, j], sem_o.at[b])

    def wait_out(j):
        b = j % NBT
        pltpu.make_async_copy(tok_v.at[b], out_hbm.at[w, j], sem_o.at[b]).wait()

    pltpu.sync_copy(pos_hbm.at[pl.ds(2 * s, 2)], pos_sh.at[pl.ds(2 * s, 2)])
    idx_cp.wait()
    plsc.subcore_barrier()
    for j in range(PDIST):
        start_prefill(j)
    for j in range(GDIST):
        wait_prefill(j)
        start_gather(j)

    waited_out = -1
    for j in range(NCHUNK):
        wait_gather(j)
        start_out(j)
        if j + PDIST < NCHUNK:
            if j >= 1:
                wait_out(j - 1)
                waited_out = j - 1
            start_prefill(j + PDIST)
        if j + GDIST < NCHUNK:
            wait_prefill(j + GDIST)
            start_gather(j + GDIST)
    for j in range(waited_out + 1, NCHUNK):
        wait_out(j)


@jax.jit
def kernel(x, token_table, pos_table):
    x3 = x.astype(jnp.int32).reshape(B, NCHUNK, CHUNK)
    pos3 = pos_table.reshape(NCHUNK, CHUNK, D)
    mesh = plsc.VectorSubcoreMesh(core_axis_name="c", subcore_axis_name="s")
    out = pl.kernel(
        _emb_body,
        out_type=jax.ShapeDtypeStruct((B, NCHUNK, CHUNK, D), jnp.float32),
        mesh=mesh,
        scratch_types=[
            pltpu.VMEM((NCHUNK, CHUNK), jnp.int32),
            pltpu.VMEM((NBT, CHUNK, D), jnp.float32),
            pltpu.VMEM_SHARED((NCHUNK, CHUNK, D), jnp.float32),
            pltpu.SemaphoreType.DMA((NBT,)),
            pltpu.SemaphoreType.DMA((NBT,)),
            pltpu.SemaphoreType.DMA((NBT,)),
            pltpu.SemaphoreType.DMA,
        ],
    )(x3, token_table, pos3)
    return out.reshape(B, MAXLEN, D)


# speedup vs baseline: 1.0138x; 1.0138x over previous
import jax
import jax.numpy as jnp
from jax import lax
from jax.experimental import pallas as pl
from jax.experimental.pallas import tpu as pltpu
from jax.experimental.pallas import tpu_sc as plsc

B = 32
MAXLEN = 2048
D = 128
CHUNK = 128
NCHUNK = MAXLEN // CHUNK
NC = 2
NS = 16
NBT = 6
GDIST = 3
PDIST = 5


def _emb_body(x_hbm, tok_hbm, pos_hbm, out_hbm, idx_v, tok_v, pos_sh,
              sem_g, sem_p, sem_o, sem_i):
    c = lax.axis_index("c")
    s = lax.axis_index("s")
    w = s * NC + c

    idx_cp = pltpu.async_copy(x_hbm.at[w], idx_v, sem_i)

    def start_prefill(j):
        b = j % NBT
        pltpu.async_copy(pos_sh.at[j], tok_v.at[b], sem_p.at[b])

    def wait_prefill(j):
        b = j % NBT
        pltpu.make_async_copy(pos_sh.at[j], tok_v.at[b], sem_p.at[b]).wait()

    def start_gather(j):
        b = j % NBT
        pltpu.async_copy(tok_hbm.at[idx_v.at[j]], tok_v.at[b], sem_g.at[b], add=True)

    def wait_gather(j):
        b = j % NBT
        pltpu.make_async_copy(tok_hbm.at[idx_v.at[j]], tok_v.at[b], sem_g.at[b]).wait()

    def start_out_pair(j):
        b = j % NBT
        pltpu.async_copy(tok_v.at[pl.ds(b, 2)], out_hbm.at[w, pl.ds(j, 2)], sem_o.at[b // 2])

    def wait_out_pair(j):
        b = j % NBT
        pltpu.make_async_copy(tok_v.at[pl.ds(b, 2)], out_hbm.at[w, pl.ds(j, 2)], sem_o.at[b // 2]).wait()

    pltpu.sync_copy(pos_hbm.at[s], pos_sh.at[s])
    idx_cp.wait()
    plsc.subcore_barrier()
    for j in range(4):
        start_prefill(j)
    for j in range(GDIST):
        wait_prefill(j)
        start_gather(j)

    waited_pair = -1
    for j in range(NCHUNK):
        wait_gather(j)
        if j % 2 == 1:
            start_out_pair(j - 1)
        if j % 2 == 0 and j + 5 < NCHUNK:
            if j >= 2:
                wait_out_pair(j - 2)
                waited_pair = j - 2
            start_prefill(j + 4)
            start_prefill(j + 5)
        if j + GDIST < NCHUNK:
            wait_prefill(j + GDIST)
            start_gather(j + GDIST)
    for j in range(waited_pair + 2, NCHUNK, 2):
        wait_out_pair(j)


@jax.jit
def kernel(x, token_table, pos_table):
    x3 = x.astype(jnp.int32).reshape(B, NCHUNK, CHUNK)
    pos3 = pos_table.reshape(NCHUNK, CHUNK, D)
    mesh = plsc.VectorSubcoreMesh(core_axis_name="c", subcore_axis_name="s")
    out = pl.kernel(
        _emb_body,
        out_type=jax.ShapeDtypeStruct((B, NCHUNK, CHUNK, D), jnp.float32),
        mesh=mesh,
        scratch_types=[
            pltpu.VMEM((NCHUNK, CHUNK), jnp.int32),
            pltpu.VMEM((NBT, CHUNK, D), jnp.float32),
            pltpu.VMEM_SHARED((NCHUNK, CHUNK, D), jnp.float32),
            pltpu.SemaphoreType.DMA((NBT,)),
            pltpu.SemaphoreType.DMA((NBT,)),
            pltpu.SemaphoreType.DMA((NBT,)),
            pltpu.SemaphoreType.DMA,
        ],
    )(x3, token_table, pos3)
    return out.reshape(B, MAXLEN, D)


# speedup vs baseline: 1.0308x; 1.0168x over previous
import jax
import jax.numpy as jnp
from jax import lax
from jax.experimental import pallas as pl
from jax.experimental.pallas import tpu as pltpu
from jax.experimental.pallas import tpu_sc as plsc

B = 32
MAXLEN = 2048
D = 128
CHUNK = 128
NCHUNK = MAXLEN // CHUNK
NC = 2
NS = 16
NBT = 6
GDIST = 4
PDIST = 5


def _emb_body(x_hbm, tok_hbm, pos_hbm, out_hbm, idx_v, tok_v, pos_sh,
              sem_g, sem_p, sem_o, sem_i):
    c = lax.axis_index("c")
    s = lax.axis_index("s")
    w = s * NC + c

    idx_cp = pltpu.async_copy(x_hbm.at[w], idx_v, sem_i)

    def start_prefill(j):
        b = j % NBT
        pltpu.async_copy(pos_sh.at[j], tok_v.at[b], sem_p.at[b])

    def wait_prefill(j):
        b = j % NBT
        pltpu.make_async_copy(pos_sh.at[j], tok_v.at[b], sem_p.at[b]).wait()

    def start_gather(j):
        b = j % NBT
        pltpu.async_copy(tok_hbm.at[idx_v.at[j]], tok_v.at[b], sem_g.at[b], add=True)

    def wait_gather(j):
        b = j % NBT
        pltpu.make_async_copy(tok_hbm.at[idx_v.at[j]], tok_v.at[b], sem_g.at[b]).wait()

    def start_out(j):
        b = j % NBT
        pltpu.async_copy(tok_v.at[b], out_hbm.at[w, j], sem_o.at[b])

    def wait_out(j):
        b = j % NBT
        pltpu.make_async_copy(tok_v.at[b], out_hbm.at[w, j], sem_o.at[b]).wait()

    pltpu.sync_copy(pos_hbm.at[s], pos_sh.at[s])
    idx_cp.wait()
    plsc.subcore_barrier()
    for j in range(PDIST):
        start_prefill(j)
    for j in range(GDIST):
        wait_prefill(j)
        start_gather(j)

    waited_out = -1
    for j in range(NCHUNK):
        wait_gather(j)
        start_out(j)
        if j + PDIST < NCHUNK:
            if j >= 1:
                wait_out(j - 1)
                waited_out = j - 1
            start_prefill(j + PDIST)
        if j + GDIST < NCHUNK:
            wait_prefill(j + GDIST)
            start_gather(j + GDIST)
    for j in range(waited_out + 1, NCHUNK):
        wait_out(j)


@jax.jit
def kernel(x, token_table, pos_table):
    x3 = x.astype(jnp.int32).reshape(B, NCHUNK, CHUNK)
    pos3 = pos_table.reshape(NCHUNK, CHUNK, D)
    mesh = plsc.VectorSubcoreMesh(core_axis_name="c", subcore_axis_name="s")
    out = pl.kernel(
        _emb_body,
        out_type=jax.ShapeDtypeStruct((B, NCHUNK, CHUNK, D), jnp.float32),
        mesh=mesh,
        scratch_types=[
            pltpu.VMEM((NCHUNK, CHUNK), jnp.int32),
            pltpu.VMEM((NBT, CHUNK, D), jnp.float32),
            pltpu.VMEM_SHARED((NCHUNK, CHUNK, D), jnp.float32),
            pltpu.SemaphoreType.DMA((NBT,)),
            pltpu.SemaphoreType.DMA((NBT,)),
            pltpu.SemaphoreType.DMA((NBT,)),
            pltpu.SemaphoreType.DMA,
        ],
    )(x3, token_table, pos3)
    return out.reshape(B, MAXLEN, D)
